# baseline (speedup 1.0000x reference)
import functools

import jax
import jax.numpy as jnp
from jax import lax
from jax.experimental import pallas as pl
from jax.experimental.pallas import tpu as pltpu


def _stage1_body(x_ref, wq_ref, wk_ref, bq_ref, bk_ref, vals_ref, inds_ref,
                 acc_ref, *, F, C, G, B, T):
    i = pl.program_id(0)

    @pl.when(i == 0)
    def _():
        acc_ref[...] = jnp.zeros_like(acc_ref)

    wq = wq_ref[...]
    wk = wk_ref[...]
    ws = jnp.concatenate(
        [jnp.sum(wq, axis=1, keepdims=True), jnp.sum(wk, axis=1, keepdims=True)],
        axis=1)
    valid = F - i * C
    row = lax.broadcasted_iota(jnp.int32, (C, 2), 0)
    ws = jnp.where(row < valid, ws, 0.0)
    x = x_ref[...]
    lane = lax.broadcasted_iota(jnp.int32, x.shape, 1)
    x = jnp.where(lane < valid, x, 0.0)
    acc_ref[...] += jnp.dot(x, ws, preferred_element_type=jnp.float32)

    @pl.when(i == G - 1)
    def _():
        BT = B * T
        sqv = acc_ref[:, 0:1] + jnp.sum(bq_ref[...])
        skv = acc_ref[:, 1:2] + jnp.sum(bk_ref[...])
        r0 = lax.broadcasted_iota(jnp.int32, (BT, T), 0)
        t0 = lax.broadcasted_iota(jnp.int32, (BT, T), 1)
        Mm = (r0 % T == t0).astype(jnp.float32)
        sqM = sqv * Mm
        skM = skv * Mm
        b0 = lax.broadcasted_iota(jnp.int32, (B, BT), 0)
        b1 = lax.broadcasted_iota(jnp.int32, (B, BT), 1)
        Rb = (b1 // T == b0).astype(jnp.float32)
        SQ = jnp.dot(Rb, sqM, preferred_element_type=jnp.float32)
        SK = jnp.dot(Rb, skM, preferred_element_type=jnp.float32)
        corr = lax.dot_general(SQ, SK, (((0,), (0,)), ((), ())),
                               preferred_element_type=jnp.float32)
        corr = corr * (1.0 / (B * 129.0))

        it0 = lax.broadcasted_iota(jnp.int32, (T, T), 0)
        it1 = lax.broadcasted_iota(jnp.int32, (T, T), 1)
        c = jnp.where(it0 == it1, -jnp.inf, corr)
        vs, ins = [], []
        for _sel in range(3):
            m = jnp.max(c, axis=1, keepdims=True)
            im = jnp.min(jnp.where(c == m, it1, T), axis=1, keepdims=True)
            c = jnp.where(it1 == im, -jnp.inf, c)
            vs.append(m)
            ins.append(im)
        i_min = jnp.minimum(ins[0], jnp.minimum(ins[1], ins[2]))
        i_max = jnp.maximum(ins[0], jnp.maximum(ins[1], ins[2]))
        i_mid = ins[0] + ins[1] + ins[2] - i_min - i_max

        def val_of(ix):
            return jnp.where(ix == ins[0], vs[0],
                             jnp.where(ix == ins[1], vs[1], vs[2]))

        vals_ref[...] = jnp.concatenate(
            [val_of(i_min), val_of(i_mid), val_of(i_max)], axis=1)
        inds_ref[...] = jnp.concatenate([i_min, i_mid, i_max], axis=1)


def _gather_body(idx_ref, x_ref, o_ref, sem):
    b = pl.program_id(0)
    copies = []
    for j in range(36):
        src = x_ref.at[0, pl.ds(idx_ref[j], 1), :]
        dst = o_ref.at[b, pl.ds(j, 1), :]
        cp = pltpu.make_async_copy(src, dst, sem)
        cp.start()
        copies.append(cp)
    for cp in copies:
        cp.wait()


def kernel(X, Wq, bq, Wk, bk, K):
    B, T, N, D = X.shape
    F = N * D
    BT = B * T
    C = 4096
    G = pl.cdiv(F, C)

    Xp = X.reshape(BT, F)
    bqr = bq.reshape(1, -1)
    bkr = bk.reshape(1, -1)

    body = functools.partial(_stage1_body, F=F, C=C, G=G, B=B, T=T)
    vals, inds = pl.pallas_call(
        body,
        grid=(G,),
        in_specs=[
            pl.BlockSpec((BT, C), lambda i: (0, i)),
            pl.BlockSpec((C, 256), lambda i: (i, 0)),
            pl.BlockSpec((C, 256), lambda i: (i, 0)),
            pl.BlockSpec((1, 256), lambda i: (0, 0)),
            pl.BlockSpec((1, 256), lambda i: (0, 0)),
        ],
        out_specs=[
            pl.BlockSpec((T, 3), lambda i: (0, 0)),
            pl.BlockSpec((T, 3), lambda i: (0, 0)),
        ],
        out_shape=[
            jax.ShapeDtypeStruct((T, 3), jnp.float32),
            jax.ShapeDtypeStruct((T, 3), jnp.int32),
        ],
        scratch_shapes=[pltpu.VMEM((BT, 2), jnp.float32)],
    )(Xp, Wq, Wk, bqr, bkr)

    Xv = X.reshape(B, T, F)
    idxf = inds.reshape(-1)
    grid_spec = pltpu.PrefetchScalarGridSpec(
        num_scalar_prefetch=1,
        grid=(B,),
        in_specs=[pl.BlockSpec((1, T, F), lambda b, idx: (b, 0, 0))],
        out_specs=pl.BlockSpec(memory_space=pl.ANY),
        scratch_shapes=[pltpu.SemaphoreType.DMA],
    )
    out = pl.pallas_call(
        _gather_body,
        grid_spec=grid_spec,
        out_shape=jax.ShapeDtypeStruct((B, T * 3, F), jnp.float32),
    )(idxf, Xv)
    gathered = out.reshape(B, T, 3, N, D)
    return (vals, inds, gathered)


# speedup vs baseline: 1.4964x; 1.4964x over previous
import jax
import jax.numpy as jnp
from jax import lax
from jax.experimental import pallas as pl
from jax.experimental.pallas import tpu as pltpu


def _wsum_body(wq_ref, wk_ref, o_ref):
    o_ref[...] = jnp.concatenate(
        [jnp.sum(wq_ref[...], axis=1, keepdims=True),
         jnp.sum(wk_ref[...], axis=1, keepdims=True)], axis=1)


def _sq_body(x_ref, wq3_ref, wk3_ref, oq_ref, ok_ref):
    x = x_ref[0]
    wq3 = wq3_ref[...][None]
    wk3 = wk3_ref[...][None]
    sq = jnp.sum(jnp.sum(x * wq3, axis=2, keepdims=True), axis=1, keepdims=True)
    sk = jnp.sum(jnp.sum(x * wk3, axis=2, keepdims=True), axis=1, keepdims=True)
    for t in range(x.shape[0]):
        oq_ref[0, 0, t] = sq[t, 0, 0]
        ok_ref[0, 0, t] = sk[t, 0, 0]


def _corr_body(sq_ref, sk_ref, bq_ref, bk_ref, vals_ref, inds_ref):
    B, T = sq_ref.shape
    SQ = sq_ref[...] + jnp.sum(bq_ref[...])
    SK = sk_ref[...] + jnp.sum(bk_ref[...])
    corr = lax.dot_general(SQ, SK, (((0,), (0,)), ((), ())),
                           preferred_element_type=jnp.float32)
    corr = corr * (1.0 / (B * 129.0))

    it0 = lax.broadcasted_iota(jnp.int32, (T, T), 0)
    it1 = lax.broadcasted_iota(jnp.int32, (T, T), 1)
    c = jnp.where(it0 == it1, -jnp.inf, corr)
    vs, ins = [], []
    for _sel in range(3):
        m = jnp.max(c, axis=1, keepdims=True)
        im = jnp.min(jnp.where(c == m, it1, T), axis=1, keepdims=True)
        c = jnp.where(it1 == im, -jnp.inf, c)
        vs.append(m)
        ins.append(im)
    i_min = jnp.minimum(ins[0], jnp.minimum(ins[1], ins[2]))
    i_max = jnp.maximum(ins[0], jnp.maximum(ins[1], ins[2]))
    i_mid = ins[0] + ins[1] + ins[2] - i_min - i_max

    def val_of(ix):
        return jnp.where(ix == ins[0], vs[0],
                         jnp.where(ix == ins[1], vs[1], vs[2]))

    vals_ref[...] = jnp.concatenate(
        [val_of(i_min), val_of(i_mid), val_of(i_max)], axis=1)
    inds_ref[...] = jnp.concatenate([i_min, i_mid, i_max], axis=1)


def _gather_body(idx_ref, x_ref, o_ref, sem):
    b = pl.program_id(0)
    copies = []
    for j in range(36):
        cp = pltpu.make_async_copy(
            x_ref.at[0, pl.ds(idx_ref[j], 1), :, :],
            o_ref.at[b, pl.ds(j, 1), :, :],
            sem)
        cp.start()
        copies.append(cp)
    for cp in copies:
        cp.wait()


def kernel(X, Wq, bq, Wk, bk, K):
    B, T, N, D = X.shape
    F = N * D
    C = 4096
    G = pl.cdiv(F, C)

    wsum2 = pl.pallas_call(
        _wsum_body,
        grid=(G,),
        in_specs=[
            pl.BlockSpec((C, 256), lambda i: (i, 0)),
            pl.BlockSpec((C, 256), lambda i: (i, 0)),
        ],
        out_specs=pl.BlockSpec((C, 2), lambda i: (i, 0)),
        out_shape=jax.ShapeDtypeStruct((F, 2), jnp.float32),
    )(Wq, Wk)
    w3q = wsum2[:, 0].reshape(N, D)
    w3k = wsum2[:, 1].reshape(N, D)

    sqm, skm = pl.pallas_call(
        _sq_body,
        grid=(B,),
        in_specs=[
            pl.BlockSpec((1, T, N, D), lambda b: (b, 0, 0, 0)),
            pl.BlockSpec((N, D), lambda b: (0, 0)),
            pl.BlockSpec((N, D), lambda b: (0, 0)),
        ],
        out_specs=[
            pl.BlockSpec((1, 1, T), lambda b: (b, 0, 0), memory_space=pltpu.SMEM),
            pl.BlockSpec((1, 1, T), lambda b: (b, 0, 0), memory_space=pltpu.SMEM),
        ],
        out_shape=[
            jax.ShapeDtypeStruct((B, 1, T), jnp.float32),
            jax.ShapeDtypeStruct((B, 1, T), jnp.float32),
        ],
    )(X, w3q, w3k)
    sqm = sqm.reshape(B, T)
    skm = skm.reshape(B, T)

    vals, inds = pl.pallas_call(
        _corr_body,
        in_specs=[
            pl.BlockSpec((B, T), lambda: (0, 0)),
            pl.BlockSpec((B, T), lambda: (0, 0)),
            pl.BlockSpec((1, 256), lambda: (0, 0)),
            pl.BlockSpec((1, 256), lambda: (0, 0)),
        ],
        out_specs=[
            pl.BlockSpec((T, 3), lambda: (0, 0)),
            pl.BlockSpec((T, 3), lambda: (0, 0)),
        ],
        out_shape=[
            jax.ShapeDtypeStruct((T, 3), jnp.float32),
            jax.ShapeDtypeStruct((T, 3), jnp.int32),
        ],
    )(sqm, skm, bq.reshape(1, -1), bk.reshape(1, -1))

    idxf = inds.reshape(-1)
    grid_spec = pltpu.PrefetchScalarGridSpec(
        num_scalar_prefetch=1,
        grid=(B,),
        in_specs=[pl.BlockSpec((1, T, N, D), lambda b, idx: (b, 0, 0, 0))],
        out_specs=pl.BlockSpec(memory_space=pl.ANY),
        scratch_shapes=[pltpu.SemaphoreType.DMA],
    )
    out = pl.pallas_call(
        _gather_body,
        grid_spec=grid_spec,
        out_shape=jax.ShapeDtypeStruct((B, T * 3, N, D), jnp.float32),
    )(idxf, X)
    gathered = out.reshape(B, T, 3, N, D)
    return (vals, inds, gathered)
